# Initial kernel scaffold; baseline (speedup 1.0000x reference)
import jax
import jax.numpy as jnp
from jax.experimental import pallas as pl


def kernel(points, batch_idx, imgs, intrinsics, extrinsics, W_dino):
    raise NotImplementedError("write your pallas kernel here")


# speedup vs baseline: 2.6706x; 2.6706x over previous
import jax
import jax.numpy as jnp
from jax import lax
from jax.experimental import pallas as pl
from jax.experimental.pallas import tpu as pltpu
from jax.experimental.pallas import tpu_sc as plsc

DIM = 384
N_VIEWS = 8
PATCH_GRID = 16
TABLE_ROWS = N_VIEWS * PATCH_GRID * PATCH_GRID
SENTINEL = TABLE_ROWS

P_ROWS = 8
P_COLS = 256
P_BLK = P_ROWS * P_COLS
N_OUT = 100000
N_PAD = 100352

NW = 32
B_PER_W = N_PAD // NW
CH = 112
NCH = B_PER_W // CH


def _dino_matmul_kernel(x_ref, w_ref, o_ref):
    o_ref[...] = jnp.dot(x_ref[...].astype(jnp.bfloat16),
                         w_ref[...].astype(jnp.bfloat16),
                         preferred_element_type=jnp.float32)


def _rb(t):
    return t.astype(jnp.bfloat16).astype(jnp.float32)


def _index_kernel(par_ref, x_ref, y_ref, z_ref, b_ref, o_ref):
    x = _rb(x_ref[...])
    y = _rb(y_ref[...])
    z = _rb(z_ref[...])
    bidx = b_ref[...]
    idx = jnp.full(x.shape, SENTINEL, jnp.int32)
    for v8 in range(N_VIEWS):
        e = [_rb(par_ref[v8, i]) for i in range(12)]
        k = [_rb(par_ref[v8, 12 + i]) for i in range(9)]
        xc = e[0] * x + e[1] * y + e[2] * z + e[3]
        yc = e[4] * x + e[5] * y + e[6] * z + e[7]
        zc = e[8] * x + e[9] * y + e[10] * z + e[11]
        xcb, ycb, zcb = _rb(xc), _rb(yc), _rb(zc)
        xi = k[0] * xcb + k[1] * ycb + k[2] * zcb
        yi = k[3] * xcb + k[4] * ycb + k[5] * zcb
        zi = k[6] * xcb + k[7] * ycb + k[8] * zcb
        u = xi / zi
        v = yi / zi
        valid = ((zc > 0.1) & (u >= 0.0) & (u < 224.0)
                 & (v >= 0.0) & (v < 224.0) & (bidx == (v8 // 4)))
        up = jnp.clip((u / 14.0).astype(jnp.int32), 0, PATCH_GRID - 1)
        vp = jnp.clip((v / 14.0).astype(jnp.int32), 0, PATCH_GRID - 1)
        cand = v8 * (PATCH_GRID * PATCH_GRID) + vp * PATCH_GRID + up
        idx = jnp.where(valid, cand, idx)
    o_ref[...] = idx


def _gather_body(table_hbm, idx_hbm, out_hbm, idx_v, buf0, buf1, sem0, sem1):
    wid = lax.axis_index("s") * 2 + lax.axis_index("c")
    base = wid * B_PER_W
    local_max = jnp.minimum(N_OUT - CH - base, B_PER_W - CH)
    pltpu.sync_copy(idx_hbm.at[pl.ds(base, B_PER_W)], idx_v)

    def start(c, buf, sem):
        local = jnp.minimum(c * CH, local_max)
        pltpu.async_copy(table_hbm.at[idx_v.at[pl.ds(local, CH)]], buf, sem)

    def wait(buf, sem):
        pltpu.make_async_copy(table_hbm.at[pl.ds(0, CH)], buf, sem).wait()

    def store(c, buf):
        local = jnp.minimum(c * CH, local_max)
        pltpu.sync_copy(buf, out_hbm.at[pl.ds(base + local, CH)])

    start(0, buf0, sem0)

    def body(i, carry):
        c0 = 2 * i
        start(c0 + 1, buf1, sem1)
        wait(buf0, sem0)
        store(c0, buf0)

        @pl.when(i < NCH // 2 - 1)
        def _():
            start(c0 + 2, buf0, sem0)

        wait(buf1, sem1)
        store(c0 + 1, buf1)
        return carry

    lax.fori_loop(0, NCH // 2, body, 0)


def kernel(points, batch_idx, imgs, intrinsics, extrinsics, W_dino):
    b, v, c, h, w = imgs.shape
    x = imgs.reshape(b * v, c, PATCH_GRID, 14, PATCH_GRID, 14)
    x = x.transpose(0, 2, 4, 1, 3, 5).reshape(b * v * PATCH_GRID * PATCH_GRID,
                                              c * 14 * 14)
    table = pl.pallas_call(
        _dino_matmul_kernel,
        out_shape=jax.ShapeDtypeStruct((TABLE_ROWS, DIM), jnp.float32),
    )(x, W_dino)
    table_pad = jnp.concatenate(
        [table, jnp.zeros((8, DIM), jnp.float32)], axis=0)

    n = points.shape[0]
    pad = N_PAD - n
    pts = jnp.pad(points, ((0, pad), (0, 0)))
    bi = jnp.pad(batch_idx, (0, pad))
    xs = pts[:, 0].reshape(-1, P_COLS)
    ys = pts[:, 1].reshape(-1, P_COLS)
    zs = pts[:, 2].reshape(-1, P_COLS)
    bi2 = bi.reshape(-1, P_COLS)
    params = jnp.concatenate(
        [extrinsics.reshape(N_VIEWS, 12), intrinsics.reshape(N_VIEWS, 9),
         jnp.zeros((N_VIEWS, 3), jnp.float32)], axis=1)

    grid = N_PAD // P_BLK
    blk = pl.BlockSpec((P_ROWS, P_COLS), lambda i: (i, 0))
    idx = pl.pallas_call(
        _index_kernel,
        grid=(grid,),
        in_specs=[pl.BlockSpec(memory_space=pltpu.SMEM), blk, blk, blk, blk],
        out_specs=blk,
        out_shape=jax.ShapeDtypeStruct((N_PAD // P_COLS, P_COLS), jnp.int32),
    )(params, xs, ys, zs, bi2)

    mesh = plsc.VectorSubcoreMesh(core_axis_name="c", subcore_axis_name="s")
    out = pl.kernel(
        _gather_body,
        out_type=jax.ShapeDtypeStruct((N_OUT, DIM), jnp.float32),
        mesh=mesh,
        scratch_types=[
            pltpu.VMEM((B_PER_W,), jnp.int32),
            pltpu.VMEM((CH, DIM), jnp.float32),
            pltpu.VMEM((CH, DIM), jnp.float32),
            pltpu.SemaphoreType.DMA,
            pltpu.SemaphoreType.DMA,
        ],
    )(table_pad, idx.reshape(-1))
    return out
